# baseline (speedup 1.0000x reference)
import functools

import jax
import jax.numpy as jnp
from jax import lax
from jax.experimental import pallas as pl
from jax.experimental.pallas import tpu as pltpu
from jax.experimental.pallas import tpu_sc as plsc

N_NODES = 10000
N_EDGES = 20000
NF = 32
NE = 8
NGRAPHS = 256

NCORES = 2
NSUB = 16
NW = NCORES * NSUB
E_PAD = 20480
E_RPT = E_PAD // NW
N_PAD = 12288
N_RPT = N_PAD // NW
D = 128
NS0 = 5120
NS1 = N_NODES - NS0
NOUT_SPLIT = 5248
G_SOUT = 512


@functools.lru_cache(maxsize=None)
def _make_sc_gather(n_rows_pad):
    rpt = n_rows_pad // NW
    nchunk = rpt // 128
    mesh = plsc.VectorSubcoreMesh(core_axis_name="c", subcore_axis_name="s",
                                  num_cores=NCORES, num_subcores=NSUB)

    @functools.partial(
        pl.kernel,
        out_type=jax.ShapeDtypeStruct((n_rows_pad, D), jnp.float32),
        mesh=mesh,
        scratch_types=[
            pltpu.VMEM((nchunk, 128), jnp.int32),
            pltpu.VMEM((rpt, D), jnp.float32),
            pltpu.SemaphoreType.DMA,
        ],
    )
    def gather_k(table_hbm, idx3_hbm, out_hbm, idx_v, rows_v, sem):
        wid = lax.axis_index("c") * NSUB + lax.axis_index("s")
        pltpu.sync_copy(idx3_hbm.at[wid], idx_v)
        for j in range(nchunk):
            pltpu.async_copy(
                table_hbm.at[idx_v.at[j]],
                rows_v.at[pl.ds(j * 128, 128)], sem).wait()
        pltpu.sync_copy(rows_v, out_hbm.at[pl.ds(wid * rpt, rpt)])

    return gather_k


@functools.lru_cache(maxsize=None)
def _make_sc_pre():
    rpt = E_RPT
    nchunk = rpt // 128
    stripe = NOUT_SPLIT // NSUB
    mesh = plsc.VectorSubcoreMesh(core_axis_name="c", subcore_axis_name="s",
                                  num_cores=NCORES, num_subcores=NSUB)

    @functools.partial(
        pl.kernel,
        out_type=(jax.ShapeDtypeStruct((E_PAD, D), jnp.float32),
                  jax.ShapeDtypeStruct((NCORES, NOUT_SPLIT, D), jnp.float32),
                  jax.ShapeDtypeStruct((NCORES, NOUT_SPLIT, D), jnp.float32)),
        mesh=mesh,
        scratch_types=(
            [pltpu.VMEM((128,), jnp.int32) for _ in range(2 * nchunk)]
            + [pltpu.VMEM((nchunk, 128), jnp.int32),
               pltpu.VMEM((rpt, D), jnp.float32),
               pltpu.VMEM_SHARED((NOUT_SPLIT, D), jnp.float32),
               pltpu.SemaphoreType.DMA]
        ),
    )
    def pre_k(table_hbm, src3_hbm, cidxA_hbm, cidxB_hbm, ones_hbm, zeros_hbm,
              xs_hbm, cntA_hbm, cntB_hbm, *scratch):
        idxA = scratch[:nchunk]
        idxB = scratch[nchunk:2 * nchunk]
        gidx_v = scratch[2 * nchunk]
        buf_v = scratch[2 * nchunk + 1]
        acc_sh = scratch[2 * nchunk + 2]
        sem = scratch[2 * nchunk + 3]
        c = lax.axis_index("c")
        s = lax.axis_index("s")
        wid = c * NSUB + s
        pltpu.sync_copy(zeros_hbm, acc_sh.at[pl.ds(s * stripe, stripe)])
        pltpu.sync_copy(ones_hbm, buf_v)
        for j in range(nchunk):
            pltpu.sync_copy(cidxA_hbm.at[wid * nchunk + j], idxA[j])
            pltpu.sync_copy(cidxB_hbm.at[wid * nchunk + j], idxB[j])
        plsc.subcore_barrier()
        for j in range(nchunk):
            pltpu.sync_copy(buf_v.at[pl.ds(j * 128, 128)],
                            acc_sh.at[idxA[j]], add=True)
        plsc.subcore_barrier()
        pltpu.sync_copy(acc_sh.at[pl.ds(s * stripe, stripe)],
                        cntA_hbm.at[c, pl.ds(s * stripe, stripe)])
        pltpu.sync_copy(zeros_hbm, acc_sh.at[pl.ds(s * stripe, stripe)])
        plsc.subcore_barrier()
        for j in range(nchunk):
            pltpu.sync_copy(buf_v.at[pl.ds(j * 128, 128)],
                            acc_sh.at[idxB[j]], add=True)
        plsc.subcore_barrier()
        pltpu.sync_copy(acc_sh.at[pl.ds(s * stripe, stripe)],
                        cntB_hbm.at[c, pl.ds(s * stripe, stripe)])
        pltpu.sync_copy(src3_hbm.at[wid], gidx_v)
        for j in range(nchunk):
            pltpu.async_copy(table_hbm.at[gidx_v.at[j]],
                             buf_v.at[pl.ds(j * 128, 128)], sem).wait()
        pltpu.sync_copy(buf_v, xs_hbm.at[pl.ds(wid * rpt, rpt)])

    return pre_k


@functools.lru_cache(maxsize=None)
def _make_sc_scatter2(n_rows_pad, n_out):
    rpt = n_rows_pad // NW
    nchunk = rpt // 128
    stripe = n_out // NSUB
    assert stripe % 8 == 0
    mesh = plsc.VectorSubcoreMesh(core_axis_name="c", subcore_axis_name="s",
                                  num_cores=NCORES, num_subcores=NSUB)

    @functools.partial(
        pl.kernel,
        out_type=(jax.ShapeDtypeStruct((NCORES, n_out, D), jnp.float32),
                  jax.ShapeDtypeStruct((NCORES, n_out, D), jnp.float32)),
        mesh=mesh,
        scratch_types=(
            [pltpu.VMEM((128,), jnp.int32) for _ in range(2 * nchunk)]
            + [pltpu.VMEM((rpt, D), jnp.float32),
               pltpu.VMEM_SHARED((n_out, D), jnp.float32)]
        ),
    )
    def scatter2_k(vals_hbm, idxA_hbm, idxB_hbm, zeros_hbm,
                   outA_hbm, outB_hbm, *scratch):
        idxA = scratch[:nchunk]
        idxB = scratch[nchunk:2 * nchunk]
        vals_v = scratch[2 * nchunk]
        acc_sh = scratch[2 * nchunk + 1]
        c = lax.axis_index("c")
        s = lax.axis_index("s")
        wid = c * NSUB + s
        pltpu.sync_copy(zeros_hbm, acc_sh.at[pl.ds(s * stripe, stripe)])
        pltpu.sync_copy(vals_hbm.at[pl.ds(wid * rpt, rpt)], vals_v)
        for j in range(nchunk):
            pltpu.sync_copy(idxA_hbm.at[wid * nchunk + j], idxA[j])
            pltpu.sync_copy(idxB_hbm.at[wid * nchunk + j], idxB[j])
        plsc.subcore_barrier()
        for j in range(nchunk):
            pltpu.sync_copy(vals_v.at[pl.ds(j * 128, 128)],
                            acc_sh.at[idxA[j]], add=True)
        plsc.subcore_barrier()
        pltpu.sync_copy(acc_sh.at[pl.ds(s * stripe, stripe)],
                        outA_hbm.at[c, pl.ds(s * stripe, stripe)])
        pltpu.sync_copy(zeros_hbm, acc_sh.at[pl.ds(s * stripe, stripe)])
        plsc.subcore_barrier()
        for j in range(nchunk):
            pltpu.sync_copy(vals_v.at[pl.ds(j * 128, 128)],
                            acc_sh.at[idxB[j]], add=True)
        plsc.subcore_barrier()
        pltpu.sync_copy(acc_sh.at[pl.ds(s * stripe, stripe)],
                        outB_hbm.at[c, pl.ds(s * stripe, stripe)])

    return scatter2_k


@functools.lru_cache(maxsize=None)
def _make_sc_graph():
    rpt = N_RPT
    nchunk = rpt // 128
    stripe = G_SOUT // NSUB
    mesh = plsc.VectorSubcoreMesh(core_axis_name="c", subcore_axis_name="s",
                                  num_cores=NCORES, num_subcores=NSUB)

    @functools.partial(
        pl.kernel,
        out_type=(jax.ShapeDtypeStruct((NCORES, G_SOUT, D), jnp.float32),
                  jax.ShapeDtypeStruct((NCORES, G_SOUT, D), jnp.float32)),
        mesh=mesh,
        scratch_types=(
            [pltpu.VMEM((128,), jnp.int32) for _ in range(nchunk)]
            + [pltpu.VMEM((rpt, D), jnp.float32),
               pltpu.VMEM((rpt, D), jnp.float32),
               pltpu.VMEM_SHARED((G_SOUT, D), jnp.float32),
               pltpu.VMEM_SHARED((G_SOUT, D), jnp.float32)]
        ),
    )
    def graph_k(vals_hbm, idx_hbm, ones_hbm, zeros_hbm,
                gp_hbm, gcnt_hbm, *scratch):
        idxs = scratch[:nchunk]
        vals_v = scratch[nchunk]
        ones_v = scratch[nchunk + 1]
        accP = scratch[nchunk + 2]
        accC = scratch[nchunk + 3]
        c = lax.axis_index("c")
        s = lax.axis_index("s")
        wid = c * NSUB + s
        pltpu.sync_copy(zeros_hbm, accP.at[pl.ds(s * stripe, stripe)])
        pltpu.sync_copy(zeros_hbm, accC.at[pl.ds(s * stripe, stripe)])
        pltpu.sync_copy(vals_hbm.at[pl.ds(wid * rpt, rpt)], vals_v)
        pltpu.sync_copy(ones_hbm, ones_v)
        for j in range(nchunk):
            pltpu.sync_copy(idx_hbm.at[wid * nchunk + j], idxs[j])
        plsc.subcore_barrier()
        for j in range(nchunk):
            pltpu.sync_copy(vals_v.at[pl.ds(j * 128, 128)],
                            accP.at[idxs[j]], add=True)
            pltpu.sync_copy(ones_v.at[pl.ds(j * 128, 128)],
                            accC.at[idxs[j]], add=True)
        plsc.subcore_barrier()
        pltpu.sync_copy(accP.at[pl.ds(s * stripe, stripe)],
                        gp_hbm.at[c, pl.ds(s * stripe, stripe)])
        pltpu.sync_copy(accC.at[pl.ds(s * stripe, stripe)],
                        gcnt_hbm.at[c, pl.ds(s * stripe, stripe)])

    return graph_k


@functools.lru_cache(maxsize=None)
def _make_sc_scatter(n_rows_pad, n_out):
    rpt = n_rows_pad // NW
    nchunk = rpt // 128
    stripe = n_out // NSUB
    assert stripe % 8 == 0
    mesh = plsc.VectorSubcoreMesh(core_axis_name="c", subcore_axis_name="s",
                                  num_cores=NCORES, num_subcores=NSUB)

    @functools.partial(
        pl.kernel,
        out_type=jax.ShapeDtypeStruct((NCORES, n_out, D), jnp.float32),
        mesh=mesh,
        scratch_types=(
            [pltpu.VMEM((128,), jnp.int32) for _ in range(nchunk)]
            + [pltpu.VMEM((rpt, D), jnp.float32),
               pltpu.VMEM_SHARED((n_out, D), jnp.float32)]
        ),
    )
    def scatter_k(vals_hbm, idx2d_hbm, zeros_hbm, out_hbm, *scratch):
        idx_vs = scratch[:nchunk]
        vals_v = scratch[nchunk]
        acc_sh = scratch[nchunk + 1]
        c = lax.axis_index("c")
        s = lax.axis_index("s")
        wid = c * NSUB + s
        pltpu.sync_copy(zeros_hbm, acc_sh.at[pl.ds(s * stripe, stripe)])
        pltpu.sync_copy(vals_hbm.at[pl.ds(wid * rpt, rpt)], vals_v)
        for j in range(nchunk):
            pltpu.sync_copy(idx2d_hbm.at[wid * nchunk + j], idx_vs[j])
        plsc.subcore_barrier()
        for j in range(nchunk):
            pltpu.sync_copy(
                vals_v.at[pl.ds(j * 128, 128)],
                acc_sh.at[idx_vs[j]], add=True)
        plsc.subcore_barrier()
        pltpu.sync_copy(
            acc_sh.at[pl.ds(s * stripe, stripe)],
            out_hbm.at[c, pl.ds(s * stripe, stripe)])

    return scatter_k


def _make_tc_edge_msg(in_ch, out_ch, kk, e_blk):
    n_steps = E_PAD // e_blk
    io = in_ch * out_ch
    n_groups = io // D
    n_fold = D // out_ch

    def body(ea_ref, xs_ref, w1_ref, b1_ref, w2_ref, b2r_ref, rx_ref, o_ref):
        h = jnp.maximum(
            jnp.dot(ea_ref[...], w1_ref[...],
                    preferred_element_type=jnp.float32) + b1_ref[...], 0.0)
        t = jnp.dot(h, w2_ref[...], preferred_element_type=jnp.float32)
        xs = xs_ref[:, 0:in_ch]
        f = jnp.dot(xs, rx_ref[...], preferred_element_type=jnp.float32)
        p = f * t
        g = p[:, 0:D]
        for c in range(1, n_groups):
            g = g + p[:, c * D:(c + 1) * D]
        acc = g[:, 0:out_ch]
        for j in range(1, n_fold):
            acc = acc + g[:, j * out_ch:(j + 1) * out_ch]
        acc = acc + jnp.dot(xs, b2r_ref[...],
                            preferred_element_type=jnp.float32)
        row = (pl.program_id(0) * e_blk
               + lax.broadcasted_iota(jnp.int32, (e_blk, 1), 0))
        acc = jnp.where(row < N_EDGES, acc, 0.0)
        if out_ch < D:
            acc = jnp.concatenate(
                [acc, jnp.zeros((e_blk, D - out_ch), jnp.float32)], axis=1)
        o_ref[...] = acc

    def call(ea, xs, w1, b1, w2, b2):
        b2r = b2.reshape(in_ch, out_ch)
        rx = jnp.kron(jnp.eye(in_ch, dtype=jnp.float32),
                      jnp.ones((1, out_ch), jnp.float32))
        return pl.pallas_call(
            body,
            grid=(n_steps,),
            in_specs=[
                pl.BlockSpec((e_blk, NE), lambda i: (i, 0)),
                pl.BlockSpec((e_blk, D), lambda i: (i, 0)),
                pl.BlockSpec((NE, kk), lambda i: (0, 0)),
                pl.BlockSpec((1, kk), lambda i: (0, 0)),
                pl.BlockSpec((kk, io), lambda i: (0, 0)),
                pl.BlockSpec((in_ch, out_ch), lambda i: (0, 0)),
                pl.BlockSpec((in_ch, io), lambda i: (0, 0)),
            ],
            out_specs=pl.BlockSpec((e_blk, D), lambda i: (i, 0)),
            out_shape=jax.ShapeDtypeStruct((E_PAD, D), jnp.float32),
        )(ea, xs, w1, b1, w2, b2r, rx)

    return call


def _tc_node_update(spA, spB, cntA, cntB, x, in_ch, root, bias, g, b,
                    attw, attb):
    out_ch = root.shape[1]

    def body(spA_ref, spB_ref, cA_ref, cB_ref, x_ref, root_ref, bias_ref,
             g_ref, b_ref, attw_ref, attb_ref, o_ref):
        s = jnp.concatenate(
            [spA_ref[0, 0:NS0, 0:out_ch] + spA_ref[1, 0:NS0, 0:out_ch],
             spB_ref[0, 0:NS1, 0:out_ch] + spB_ref[1, 0:NS1, 0:out_ch]],
            axis=0)
        cnt = jnp.concatenate(
            [cA_ref[0, 0:NS0, 0:1] + cA_ref[1, 0:NS0, 0:1],
             cB_ref[0, 0:NS1, 0:1] + cB_ref[1, 0:NS1, 0:1]], axis=0)
        agg = s / jnp.maximum(cnt, 1.0)
        xin = x_ref[:, 0:in_ch]
        hp = agg + jnp.dot(xin, root_ref[...],
                           preferred_element_type=jnp.float32) + bias_ref[...]
        m = jnp.mean(hp, axis=0, keepdims=True)
        hc = hp - m
        v = jnp.mean(hc * hc, axis=0, keepdims=True)
        hb = g_ref[...] * hc / jnp.sqrt(v + 1e-5) + b_ref[...]
        hr = jnp.maximum(hb, 0.0)
        a = jnp.dot(hr, attw_ref[...],
                    preferred_element_type=jnp.float32) + attb_ref[...]
        sig = 1.0 / (1.0 + jnp.exp(-a))
        out = hr * sig
        if out_ch < D:
            out = jnp.concatenate(
                [out, jnp.zeros((N_NODES, D - out_ch), jnp.float32)], axis=1)
        o_ref[...] = out

    return pl.pallas_call(
        body,
        out_shape=jax.ShapeDtypeStruct((N_NODES, D), jnp.float32),
    )(spA, spB, cntA, cntB, x, root, bias.reshape(1, out_ch),
      g.reshape(1, out_ch), b.reshape(1, out_ch), attw, attb.reshape(1, 1))


def _tc_pool_mlp(gp, gcnt, fc1w, fc1b, fc2w, fc2b):
    def body(gp_ref, gc_ref, w1_ref, b1_ref, w2_ref, b2_ref, o_ref):
        s = gp_ref[0, 0:NGRAPHS, 0:NF] + gp_ref[1, 0:NGRAPHS, 0:NF]
        cnt = gc_ref[0, 0:NGRAPHS, 0:1] + gc_ref[1, 0:NGRAPHS, 0:1]
        gmean = s / jnp.maximum(cnt, 1.0)
        h1 = jnp.maximum(
            jnp.dot(gmean, w1_ref[...],
                    preferred_element_type=jnp.float32) + b1_ref[...], 0.0)
        o_ref[...] = jnp.dot(h1, w2_ref[...],
                             preferred_element_type=jnp.float32) + b2_ref[...]

    return pl.pallas_call(
        body,
        out_shape=jax.ShapeDtypeStruct((NGRAPHS, 1), jnp.float32),
    )(gp, gcnt, fc1w, fc1b.reshape(1, -1), fc2w, fc2b.reshape(1, 1))


_edge1 = _make_tc_edge_msg(NF, 128, 128, 512)
_edge2 = _make_tc_edge_msg(128, 64, 128, 256)
_edge3 = _make_tc_edge_msg(64, 32, 64, 512)


def _nnconv_layer(edge_call, xs, x_raw, in_ch, dstA, dstB,
                  cntA, cntB, z_node, ea_p, p, pfx, bn_pfx, att_pfx):
    msg = edge_call(ea_p, xs, p[pfx + '_w1'], p[pfx + '_b1'].reshape(1, -1),
                    p[pfx + '_w2'], p[pfx + '_b2'].reshape(1, -1))
    spA, spB = _make_sc_scatter2(E_PAD, NOUT_SPLIT)(msg, dstA, dstB, z_node)
    cpfx = pfx.replace('nn', 'conv')
    return _tc_node_update(spA, spB, cntA, cntB, x_raw, in_ch,
                           p[cpfx + '_root'], p[cpfx + '_bias'],
                           p[bn_pfx + '_g'], p[bn_pfx + '_b'],
                           p[att_pfx + '_w'], p[att_pfx + '_b'])


def kernel(x, edge_index, edge_attr, batch, params):
    p = params
    src = edge_index[0]
    dst = edge_index[1]

    epad = E_PAD - N_EDGES
    src3 = jnp.pad(src, (0, epad)).reshape(NW, E_RPT // 128, 128)
    dst_p = jnp.pad(dst, (0, epad))
    emask_b = jnp.arange(E_PAD, dtype=jnp.int32) < N_EDGES
    dstA = jnp.where(dst_p < NS0, dst_p, NS0).reshape(E_PAD // 128, 128)
    dstB = jnp.where(dst_p >= NS0, dst_p - NS0, NS0).reshape(E_PAD // 128, 128)
    cdstA = jnp.where(emask_b & (dst_p < NS0),
                      dst_p, NS0).reshape(E_PAD // 128, 128)
    cdstB = jnp.where(emask_b & (dst_p >= NS0),
                      dst_p - NS0, NS0).reshape(E_PAD // 128, 128)
    ea_p = jnp.pad(edge_attr, ((0, epad), (0, 0)))

    npad = N_PAD - N_NODES
    batch2d = jnp.pad(batch, (0, npad),
                      constant_values=NGRAPHS).reshape(N_PAD // 128, 128)

    ones_e = jnp.ones((E_RPT, D), jnp.float32)
    ones_n = jnp.ones((N_RPT, D), jnp.float32)
    z_node = jnp.zeros((NOUT_SPLIT // NSUB, D), jnp.float32)
    z_g = jnp.zeros((G_SOUT // NSUB, D), jnp.float32)

    x_p128 = jnp.pad(x, ((0, 0), (0, D - NF)))
    xs1, cntA, cntB = _make_sc_pre()(x_p128, src3, cdstA, cdstB,
                                     ones_e, z_node)

    h1 = _nnconv_layer(_edge1, xs1, x, NF, dstA, dstB, cntA, cntB,
                       z_node, ea_p, p, 'nn1', 'bn1', 'att1')
    xs2 = _make_sc_gather(E_PAD)(h1, src3)
    h2 = _nnconv_layer(_edge2, xs2, h1, 128, dstA, dstB, cntA, cntB,
                       z_node, ea_p, p, 'nn2', 'bn2', 'att2')
    xs3 = _make_sc_gather(E_PAD)(h2, src3)
    h3 = _nnconv_layer(_edge3, xs3, h2, 64, dstA, dstB, cntA, cntB,
                       z_node, ea_p, p, 'nn3', 'bn3', 'att3')

    h3_p = jnp.pad(h3, ((0, npad), (0, 0)))
    gp, gcnt = _make_sc_graph()(h3_p, batch2d, ones_n, z_g)
    return _tc_pool_mlp(gp, gcnt, p['fc1_w'], p['fc1_b'],
                        p['fc2_w'], p['fc2_b'])


# speedup vs baseline: 1.0319x; 1.0319x over previous
import functools

import jax
import jax.numpy as jnp
from jax import lax
from jax.experimental import pallas as pl
from jax.experimental.pallas import tpu as pltpu
from jax.experimental.pallas import tpu_sc as plsc

N_NODES = 10000
N_EDGES = 20000
NF = 32
NE = 8
NGRAPHS = 256

NCORES = 2
NSUB = 16
NW = NCORES * NSUB
E_PAD = 20480
E_RPT = E_PAD // NW
N_PAD = 12288
N_RPT = N_PAD // NW
D = 128
NS0 = 5120
NS1 = N_NODES - NS0
NOUT_SPLIT = 5248
G_SOUT = 512


@functools.lru_cache(maxsize=None)
def _make_sc_gather(n_rows_pad):
    rpt = n_rows_pad // NW
    nchunk = rpt // 128
    mesh = plsc.VectorSubcoreMesh(core_axis_name="c", subcore_axis_name="s",
                                  num_cores=NCORES, num_subcores=NSUB)

    @functools.partial(
        pl.kernel,
        out_type=jax.ShapeDtypeStruct((n_rows_pad, D), jnp.float32),
        mesh=mesh,
        scratch_types=[
            pltpu.VMEM((nchunk, 128), jnp.int32),
            pltpu.VMEM((rpt, D), jnp.float32),
            pltpu.SemaphoreType.DMA,
        ],
    )
    def gather_k(table_hbm, idx3_hbm, out_hbm, idx_v, rows_v, sem):
        wid = lax.axis_index("c") * NSUB + lax.axis_index("s")
        pltpu.sync_copy(idx3_hbm.at[wid], idx_v)
        pend = [pltpu.async_copy(table_hbm.at[idx_v.at[j]],
                                 rows_v.at[pl.ds(j * 128, 128)], sem)
                for j in range(nchunk)]
        for d in pend:
            d.wait()
        pltpu.sync_copy(rows_v, out_hbm.at[pl.ds(wid * rpt, rpt)])

    return gather_k


@functools.lru_cache(maxsize=None)
def _make_sc_pre():
    rpt = E_RPT
    nchunk = rpt // 128
    stripe = NOUT_SPLIT // NSUB
    mesh = plsc.VectorSubcoreMesh(core_axis_name="c", subcore_axis_name="s",
                                  num_cores=NCORES, num_subcores=NSUB)

    @functools.partial(
        pl.kernel,
        out_type=(jax.ShapeDtypeStruct((E_PAD, D), jnp.float32),
                  jax.ShapeDtypeStruct((NCORES, NOUT_SPLIT, D), jnp.float32),
                  jax.ShapeDtypeStruct((NCORES, NOUT_SPLIT, D), jnp.float32)),
        mesh=mesh,
        scratch_types=(
            [pltpu.VMEM((128,), jnp.int32) for _ in range(2 * nchunk)]
            + [pltpu.VMEM((nchunk, 128), jnp.int32),
               pltpu.VMEM((rpt, D), jnp.float32),
               pltpu.VMEM_SHARED((NOUT_SPLIT, D), jnp.float32),
               pltpu.SemaphoreType.DMA]
        ),
    )
    def pre_k(table_hbm, src3_hbm, cidxA_hbm, cidxB_hbm, ones_hbm, zeros_hbm,
              xs_hbm, cntA_hbm, cntB_hbm, *scratch):
        idxA = scratch[:nchunk]
        idxB = scratch[nchunk:2 * nchunk]
        gidx_v = scratch[2 * nchunk]
        buf_v = scratch[2 * nchunk + 1]
        acc_sh = scratch[2 * nchunk + 2]
        sem = scratch[2 * nchunk + 3]
        c = lax.axis_index("c")
        s = lax.axis_index("s")
        wid = c * NSUB + s
        pend = [pltpu.async_copy(
            zeros_hbm, acc_sh.at[pl.ds(s * stripe, stripe)], sem),
            pltpu.async_copy(ones_hbm, buf_v, sem),
            pltpu.async_copy(src3_hbm.at[wid], gidx_v, sem)]
        for j in range(nchunk):
            pend.append(pltpu.async_copy(
                cidxA_hbm.at[wid * nchunk + j], idxA[j], sem))
            pend.append(pltpu.async_copy(
                cidxB_hbm.at[wid * nchunk + j], idxB[j], sem))
        for d in pend:
            d.wait()
        plsc.subcore_barrier()
        pend = [pltpu.async_copy(buf_v.at[pl.ds(j * 128, 128)],
                                 acc_sh.at[idxA[j]], sem, add=True)
                for j in range(nchunk)]
        for d in pend:
            d.wait()
        plsc.subcore_barrier()
        pltpu.sync_copy(acc_sh.at[pl.ds(s * stripe, stripe)],
                        cntA_hbm.at[c, pl.ds(s * stripe, stripe)])
        pltpu.sync_copy(zeros_hbm, acc_sh.at[pl.ds(s * stripe, stripe)])
        plsc.subcore_barrier()
        pend = [pltpu.async_copy(buf_v.at[pl.ds(j * 128, 128)],
                                 acc_sh.at[idxB[j]], sem, add=True)
                for j in range(nchunk)]
        for d in pend:
            d.wait()
        plsc.subcore_barrier()
        pltpu.sync_copy(acc_sh.at[pl.ds(s * stripe, stripe)],
                        cntB_hbm.at[c, pl.ds(s * stripe, stripe)])
        pend = [pltpu.async_copy(table_hbm.at[gidx_v.at[j]],
                                 buf_v.at[pl.ds(j * 128, 128)], sem)
                for j in range(nchunk)]
        for d in pend:
            d.wait()
        pltpu.sync_copy(buf_v, xs_hbm.at[pl.ds(wid * rpt, rpt)])

    return pre_k


@functools.lru_cache(maxsize=None)
def _make_sc_scatter2(n_rows_pad, n_out):
    rpt = n_rows_pad // NW
    nchunk = rpt // 128
    stripe = n_out // NSUB
    assert stripe % 8 == 0
    mesh = plsc.VectorSubcoreMesh(core_axis_name="c", subcore_axis_name="s",
                                  num_cores=NCORES, num_subcores=NSUB)

    @functools.partial(
        pl.kernel,
        out_type=(jax.ShapeDtypeStruct((NCORES, n_out, D), jnp.float32),
                  jax.ShapeDtypeStruct((NCORES, n_out, D), jnp.float32)),
        mesh=mesh,
        scratch_types=(
            [pltpu.VMEM((128,), jnp.int32) for _ in range(2 * nchunk)]
            + [pltpu.VMEM((rpt, D), jnp.float32),
               pltpu.VMEM_SHARED((n_out, D), jnp.float32),
               pltpu.SemaphoreType.DMA]
        ),
    )
    def scatter2_k(vals_hbm, idxA_hbm, idxB_hbm, zeros_hbm,
                   outA_hbm, outB_hbm, *scratch):
        idxA = scratch[:nchunk]
        idxB = scratch[nchunk:2 * nchunk]
        vals_v = scratch[2 * nchunk]
        acc_sh = scratch[2 * nchunk + 1]
        sem = scratch[2 * nchunk + 2]
        c = lax.axis_index("c")
        s = lax.axis_index("s")
        wid = c * NSUB + s
        pend = [pltpu.async_copy(
            zeros_hbm, acc_sh.at[pl.ds(s * stripe, stripe)], sem)]
        pend.append(pltpu.async_copy(
            vals_hbm.at[pl.ds(wid * rpt, rpt)], vals_v, sem))
        for j in range(nchunk):
            pend.append(pltpu.async_copy(
                idxA_hbm.at[wid * nchunk + j], idxA[j], sem))
            pend.append(pltpu.async_copy(
                idxB_hbm.at[wid * nchunk + j], idxB[j], sem))
        for d in pend:
            d.wait()
        plsc.subcore_barrier()
        pend = [pltpu.async_copy(vals_v.at[pl.ds(j * 128, 128)],
                                 acc_sh.at[idxA[j]], sem, add=True)
                for j in range(nchunk)]
        for d in pend:
            d.wait()
        plsc.subcore_barrier()
        pltpu.sync_copy(acc_sh.at[pl.ds(s * stripe, stripe)],
                        outA_hbm.at[c, pl.ds(s * stripe, stripe)])
        pltpu.sync_copy(zeros_hbm, acc_sh.at[pl.ds(s * stripe, stripe)])
        plsc.subcore_barrier()
        pend = [pltpu.async_copy(vals_v.at[pl.ds(j * 128, 128)],
                                 acc_sh.at[idxB[j]], sem, add=True)
                for j in range(nchunk)]
        for d in pend:
            d.wait()
        plsc.subcore_barrier()
        pltpu.sync_copy(acc_sh.at[pl.ds(s * stripe, stripe)],
                        outB_hbm.at[c, pl.ds(s * stripe, stripe)])

    return scatter2_k


@functools.lru_cache(maxsize=None)
def _make_sc_graph():
    rpt = N_RPT
    nchunk = rpt // 128
    stripe = G_SOUT // NSUB
    mesh = plsc.VectorSubcoreMesh(core_axis_name="c", subcore_axis_name="s",
                                  num_cores=NCORES, num_subcores=NSUB)

    @functools.partial(
        pl.kernel,
        out_type=(jax.ShapeDtypeStruct((NCORES, G_SOUT, D), jnp.float32),
                  jax.ShapeDtypeStruct((NCORES, G_SOUT, D), jnp.float32)),
        mesh=mesh,
        scratch_types=(
            [pltpu.VMEM((128,), jnp.int32) for _ in range(nchunk)]
            + [pltpu.VMEM((rpt, D), jnp.float32),
               pltpu.VMEM((rpt, D), jnp.float32),
               pltpu.VMEM_SHARED((G_SOUT, D), jnp.float32),
               pltpu.VMEM_SHARED((G_SOUT, D), jnp.float32),
               pltpu.SemaphoreType.DMA]
        ),
    )
    def graph_k(vals_hbm, idx_hbm, ones_hbm, zeros_hbm,
                gp_hbm, gcnt_hbm, *scratch):
        idxs = scratch[:nchunk]
        vals_v = scratch[nchunk]
        ones_v = scratch[nchunk + 1]
        accP = scratch[nchunk + 2]
        accC = scratch[nchunk + 3]
        sem = scratch[nchunk + 4]
        c = lax.axis_index("c")
        s = lax.axis_index("s")
        wid = c * NSUB + s
        pend = [pltpu.async_copy(
            zeros_hbm, accP.at[pl.ds(s * stripe, stripe)], sem),
            pltpu.async_copy(
                zeros_hbm, accC.at[pl.ds(s * stripe, stripe)], sem),
            pltpu.async_copy(
                vals_hbm.at[pl.ds(wid * rpt, rpt)], vals_v, sem),
            pltpu.async_copy(ones_hbm, ones_v, sem)]
        for j in range(nchunk):
            pend.append(pltpu.async_copy(
                idx_hbm.at[wid * nchunk + j], idxs[j], sem))
        for d in pend:
            d.wait()
        plsc.subcore_barrier()
        pend = []
        for j in range(nchunk):
            pend.append(pltpu.async_copy(
                vals_v.at[pl.ds(j * 128, 128)], accP.at[idxs[j]], sem,
                add=True))
            pend.append(pltpu.async_copy(
                ones_v.at[pl.ds(j * 128, 128)], accC.at[idxs[j]], sem,
                add=True))
        for d in pend:
            d.wait()
        plsc.subcore_barrier()
        pltpu.sync_copy(accP.at[pl.ds(s * stripe, stripe)],
                        gp_hbm.at[c, pl.ds(s * stripe, stripe)])
        pltpu.sync_copy(accC.at[pl.ds(s * stripe, stripe)],
                        gcnt_hbm.at[c, pl.ds(s * stripe, stripe)])

    return graph_k


@functools.lru_cache(maxsize=None)
def _make_sc_scatter(n_rows_pad, n_out):
    rpt = n_rows_pad // NW
    nchunk = rpt // 128
    stripe = n_out // NSUB
    assert stripe % 8 == 0
    mesh = plsc.VectorSubcoreMesh(core_axis_name="c", subcore_axis_name="s",
                                  num_cores=NCORES, num_subcores=NSUB)

    @functools.partial(
        pl.kernel,
        out_type=jax.ShapeDtypeStruct((NCORES, n_out, D), jnp.float32),
        mesh=mesh,
        scratch_types=(
            [pltpu.VMEM((128,), jnp.int32) for _ in range(nchunk)]
            + [pltpu.VMEM((rpt, D), jnp.float32),
               pltpu.VMEM_SHARED((n_out, D), jnp.float32)]
        ),
    )
    def scatter_k(vals_hbm, idx2d_hbm, zeros_hbm, out_hbm, *scratch):
        idx_vs = scratch[:nchunk]
        vals_v = scratch[nchunk]
        acc_sh = scratch[nchunk + 1]
        c = lax.axis_index("c")
        s = lax.axis_index("s")
        wid = c * NSUB + s
        pltpu.sync_copy(zeros_hbm, acc_sh.at[pl.ds(s * stripe, stripe)])
        pltpu.sync_copy(vals_hbm.at[pl.ds(wid * rpt, rpt)], vals_v)
        for j in range(nchunk):
            pltpu.sync_copy(idx2d_hbm.at[wid * nchunk + j], idx_vs[j])
        plsc.subcore_barrier()
        for j in range(nchunk):
            pltpu.sync_copy(
                vals_v.at[pl.ds(j * 128, 128)],
                acc_sh.at[idx_vs[j]], add=True)
        plsc.subcore_barrier()
        pltpu.sync_copy(
            acc_sh.at[pl.ds(s * stripe, stripe)],
            out_hbm.at[c, pl.ds(s * stripe, stripe)])

    return scatter_k


def _make_tc_edge_msg(in_ch, out_ch, kk, e_blk):
    n_steps = E_PAD // e_blk
    io = in_ch * out_ch
    n_groups = io // D
    n_fold = D // out_ch

    def body(ea_ref, xs_ref, w1_ref, b1_ref, w2_ref, b2r_ref, rx_ref, o_ref):
        h = jnp.maximum(
            jnp.dot(ea_ref[...], w1_ref[...],
                    preferred_element_type=jnp.float32) + b1_ref[...], 0.0)
        t = jnp.dot(h, w2_ref[...], preferred_element_type=jnp.float32)
        xs = xs_ref[:, 0:in_ch]
        f = jnp.dot(xs, rx_ref[...], preferred_element_type=jnp.float32)
        p = f * t
        g = p[:, 0:D]
        for c in range(1, n_groups):
            g = g + p[:, c * D:(c + 1) * D]
        acc = g[:, 0:out_ch]
        for j in range(1, n_fold):
            acc = acc + g[:, j * out_ch:(j + 1) * out_ch]
        acc = acc + jnp.dot(xs, b2r_ref[...],
                            preferred_element_type=jnp.float32)
        row = (pl.program_id(0) * e_blk
               + lax.broadcasted_iota(jnp.int32, (e_blk, 1), 0))
        acc = jnp.where(row < N_EDGES, acc, 0.0)
        if out_ch < D:
            acc = jnp.concatenate(
                [acc, jnp.zeros((e_blk, D - out_ch), jnp.float32)], axis=1)
        o_ref[...] = acc

    def call(ea, xs, w1, b1, w2, b2):
        b2r = b2.reshape(in_ch, out_ch)
        rx = jnp.kron(jnp.eye(in_ch, dtype=jnp.float32),
                      jnp.ones((1, out_ch), jnp.float32))
        return pl.pallas_call(
            body,
            grid=(n_steps,),
            in_specs=[
                pl.BlockSpec((e_blk, NE), lambda i: (i, 0)),
                pl.BlockSpec((e_blk, D), lambda i: (i, 0)),
                pl.BlockSpec((NE, kk), lambda i: (0, 0)),
                pl.BlockSpec((1, kk), lambda i: (0, 0)),
                pl.BlockSpec((kk, io), lambda i: (0, 0)),
                pl.BlockSpec((in_ch, out_ch), lambda i: (0, 0)),
                pl.BlockSpec((in_ch, io), lambda i: (0, 0)),
            ],
            out_specs=pl.BlockSpec((e_blk, D), lambda i: (i, 0)),
            out_shape=jax.ShapeDtypeStruct((E_PAD, D), jnp.float32),
        )(ea, xs, w1, b1, w2, b2r, rx)

    return call


def _tc_node_update(spA, spB, cntA, cntB, x, in_ch, root, bias, g, b,
                    attw, attb):
    out_ch = root.shape[1]

    def body(spA_ref, spB_ref, cA_ref, cB_ref, x_ref, root_ref, bias_ref,
             g_ref, b_ref, attw_ref, attb_ref, o_ref):
        s = jnp.concatenate(
            [spA_ref[0, 0:NS0, 0:out_ch] + spA_ref[1, 0:NS0, 0:out_ch],
             spB_ref[0, 0:NS1, 0:out_ch] + spB_ref[1, 0:NS1, 0:out_ch]],
            axis=0)
        cnt = jnp.concatenate(
            [cA_ref[0, 0:NS0, 0:1] + cA_ref[1, 0:NS0, 0:1],
             cB_ref[0, 0:NS1, 0:1] + cB_ref[1, 0:NS1, 0:1]], axis=0)
        agg = s / jnp.maximum(cnt, 1.0)
        xin = x_ref[:, 0:in_ch]
        hp = agg + jnp.dot(xin, root_ref[...],
                           preferred_element_type=jnp.float32) + bias_ref[...]
        m = jnp.mean(hp, axis=0, keepdims=True)
        hc = hp - m
        v = jnp.mean(hc * hc, axis=0, keepdims=True)
        hb = g_ref[...] * hc / jnp.sqrt(v + 1e-5) + b_ref[...]
        hr = jnp.maximum(hb, 0.0)
        a = jnp.dot(hr, attw_ref[...],
                    preferred_element_type=jnp.float32) + attb_ref[...]
        sig = 1.0 / (1.0 + jnp.exp(-a))
        out = hr * sig
        if out_ch < D:
            out = jnp.concatenate(
                [out, jnp.zeros((N_NODES, D - out_ch), jnp.float32)], axis=1)
        o_ref[...] = out

    return pl.pallas_call(
        body,
        out_shape=jax.ShapeDtypeStruct((N_NODES, D), jnp.float32),
    )(spA, spB, cntA, cntB, x, root, bias.reshape(1, out_ch),
      g.reshape(1, out_ch), b.reshape(1, out_ch), attw, attb.reshape(1, 1))


def _tc_pool_mlp(gp, gcnt, fc1w, fc1b, fc2w, fc2b):
    def body(gp_ref, gc_ref, w1_ref, b1_ref, w2_ref, b2_ref, o_ref):
        s = gp_ref[0, 0:NGRAPHS, 0:NF] + gp_ref[1, 0:NGRAPHS, 0:NF]
        cnt = gc_ref[0, 0:NGRAPHS, 0:1] + gc_ref[1, 0:NGRAPHS, 0:1]
        gmean = s / jnp.maximum(cnt, 1.0)
        h1 = jnp.maximum(
            jnp.dot(gmean, w1_ref[...],
                    preferred_element_type=jnp.float32) + b1_ref[...], 0.0)
        o_ref[...] = jnp.dot(h1, w2_ref[...],
                             preferred_element_type=jnp.float32) + b2_ref[...]

    return pl.pallas_call(
        body,
        out_shape=jax.ShapeDtypeStruct((NGRAPHS, 1), jnp.float32),
    )(gp, gcnt, fc1w, fc1b.reshape(1, -1), fc2w, fc2b.reshape(1, 1))


_edge1 = _make_tc_edge_msg(NF, 128, 128, 512)
_edge2 = _make_tc_edge_msg(128, 64, 128, 256)
_edge3 = _make_tc_edge_msg(64, 32, 64, 512)


def _nnconv_layer(edge_call, xs, x_raw, in_ch, dstA, dstB,
                  cntA, cntB, z_node, ea_p, p, pfx, bn_pfx, att_pfx):
    msg = edge_call(ea_p, xs, p[pfx + '_w1'], p[pfx + '_b1'].reshape(1, -1),
                    p[pfx + '_w2'], p[pfx + '_b2'].reshape(1, -1))
    spA, spB = _make_sc_scatter2(E_PAD, NOUT_SPLIT)(msg, dstA, dstB, z_node)
    cpfx = pfx.replace('nn', 'conv')
    return _tc_node_update(spA, spB, cntA, cntB, x_raw, in_ch,
                           p[cpfx + '_root'], p[cpfx + '_bias'],
                           p[bn_pfx + '_g'], p[bn_pfx + '_b'],
                           p[att_pfx + '_w'], p[att_pfx + '_b'])


def kernel(x, edge_index, edge_attr, batch, params):
    p = params
    src = edge_index[0]
    dst = edge_index[1]

    epad = E_PAD - N_EDGES
    src3 = jnp.pad(src, (0, epad)).reshape(NW, E_RPT // 128, 128)
    dst_p = jnp.pad(dst, (0, epad))
    emask_b = jnp.arange(E_PAD, dtype=jnp.int32) < N_EDGES
    dstA = jnp.where(dst_p < NS0, dst_p, NS0).reshape(E_PAD // 128, 128)
    dstB = jnp.where(dst_p >= NS0, dst_p - NS0, NS0).reshape(E_PAD // 128, 128)
    cdstA = jnp.where(emask_b & (dst_p < NS0),
                      dst_p, NS0).reshape(E_PAD // 128, 128)
    cdstB = jnp.where(emask_b & (dst_p >= NS0),
                      dst_p - NS0, NS0).reshape(E_PAD // 128, 128)
    ea_p = jnp.pad(edge_attr, ((0, epad), (0, 0)))

    npad = N_PAD - N_NODES
    batch2d = jnp.pad(batch, (0, npad),
                      constant_values=NGRAPHS).reshape(N_PAD // 128, 128)

    ones_e = jnp.ones((E_RPT, D), jnp.float32)
    ones_n = jnp.ones((N_RPT, D), jnp.float32)
    z_node = jnp.zeros((NOUT_SPLIT // NSUB, D), jnp.float32)
    z_g = jnp.zeros((G_SOUT // NSUB, D), jnp.float32)

    x_p128 = jnp.pad(x, ((0, 0), (0, D - NF)))
    xs1, cntA, cntB = _make_sc_pre()(x_p128, src3, cdstA, cdstB,
                                     ones_e, z_node)

    h1 = _nnconv_layer(_edge1, xs1, x, NF, dstA, dstB, cntA, cntB,
                       z_node, ea_p, p, 'nn1', 'bn1', 'att1')
    xs2 = _make_sc_gather(E_PAD)(h1, src3)
    h2 = _nnconv_layer(_edge2, xs2, h1, 128, dstA, dstB, cntA, cntB,
                       z_node, ea_p, p, 'nn2', 'bn2', 'att2')
    xs3 = _make_sc_gather(E_PAD)(h2, src3)
    h3 = _nnconv_layer(_edge3, xs3, h2, 64, dstA, dstB, cntA, cntB,
                       z_node, ea_p, p, 'nn3', 'bn3', 'att3')

    h3_p = jnp.pad(h3, ((0, npad), (0, 0)))
    gp, gcnt = _make_sc_graph()(h3_p, batch2d, ones_n, z_g)
    return _tc_pool_mlp(gp, gcnt, p['fc1_w'], p['fc1_b'],
                        p['fc2_w'], p['fc2_b'])


# speedup vs baseline: 1.0416x; 1.0094x over previous
import functools

import jax
import jax.numpy as jnp
from jax import lax
from jax.experimental import pallas as pl
from jax.experimental.pallas import tpu as pltpu
from jax.experimental.pallas import tpu_sc as plsc

N_NODES = 10000
N_EDGES = 20000
NF = 32
NE = 8
NGRAPHS = 256

NCORES = 2
NSUB = 16
NW = NCORES * NSUB
E_PAD = 20480
E_RPT = E_PAD // NW
N_PAD = 12288
N_RPT = N_PAD // NW
D = 128
NS0 = 5120
NS1 = N_NODES - NS0
NOUT_SPLIT = 5248
G_SOUT = 512


@functools.lru_cache(maxsize=None)
def _make_sc_gather(n_rows_pad):
    rpt = n_rows_pad // NW
    nchunk = rpt // 128
    mesh = plsc.VectorSubcoreMesh(core_axis_name="c", subcore_axis_name="s",
                                  num_cores=NCORES, num_subcores=NSUB)

    @functools.partial(
        pl.kernel,
        out_type=jax.ShapeDtypeStruct((n_rows_pad, D), jnp.float32),
        mesh=mesh,
        scratch_types=[
            pltpu.VMEM((nchunk, 128), jnp.int32),
            pltpu.VMEM((rpt, D), jnp.float32),
            pltpu.SemaphoreType.DMA,
        ],
    )
    def gather_k(table_hbm, idx3_hbm, out_hbm, idx_v, rows_v, sem):
        wid = lax.axis_index("c") * NSUB + lax.axis_index("s")
        pltpu.sync_copy(idx3_hbm.at[wid], idx_v)
        pend = [pltpu.async_copy(table_hbm.at[idx_v.at[j]],
                                 rows_v.at[pl.ds(j * 128, 128)], sem)
                for j in range(nchunk)]
        for d in pend:
            d.wait()
        pltpu.sync_copy(rows_v, out_hbm.at[pl.ds(wid * rpt, rpt)])

    return gather_k


@functools.lru_cache(maxsize=None)
def _make_sc_pre():
    rpt = E_RPT
    nchunk = rpt // 128
    stripe = NOUT_SPLIT // NSUB
    mesh = plsc.VectorSubcoreMesh(core_axis_name="c", subcore_axis_name="s",
                                  num_cores=NCORES, num_subcores=NSUB)

    @functools.partial(
        pl.kernel,
        out_type=(jax.ShapeDtypeStruct((E_PAD, D), jnp.float32),
                  jax.ShapeDtypeStruct((NCORES, NOUT_SPLIT, D), jnp.float32),
                  jax.ShapeDtypeStruct((NCORES, NOUT_SPLIT, D), jnp.float32)),
        mesh=mesh,
        scratch_types=(
            [pltpu.VMEM((128,), jnp.int32) for _ in range(2 * nchunk)]
            + [pltpu.VMEM((nchunk, 128), jnp.int32),
               pltpu.VMEM((rpt, D), jnp.float32),
               pltpu.VMEM_SHARED((NOUT_SPLIT, D), jnp.float32),
               pltpu.SemaphoreType.DMA]
        ),
    )
    def pre_k(table_hbm, src3_hbm, cidxA_hbm, cidxB_hbm, ones_hbm, zeros_hbm,
              xs_hbm, cntA_hbm, cntB_hbm, *scratch):
        idxA = scratch[:nchunk]
        idxB = scratch[nchunk:2 * nchunk]
        gidx_v = scratch[2 * nchunk]
        buf_v = scratch[2 * nchunk + 1]
        acc_sh = scratch[2 * nchunk + 2]
        sem = scratch[2 * nchunk + 3]
        c = lax.axis_index("c")
        s = lax.axis_index("s")
        wid = c * NSUB + s
        pend = [pltpu.async_copy(
            zeros_hbm, acc_sh.at[pl.ds(s * stripe, stripe)], sem),
            pltpu.async_copy(ones_hbm, buf_v, sem),
            pltpu.async_copy(src3_hbm.at[wid], gidx_v, sem)]
        for j in range(nchunk):
            pend.append(pltpu.async_copy(
                cidxA_hbm.at[wid * nchunk + j], idxA[j], sem))
            pend.append(pltpu.async_copy(
                cidxB_hbm.at[wid * nchunk + j], idxB[j], sem))
        for d in pend:
            d.wait()
        plsc.subcore_barrier()
        pend = [pltpu.async_copy(buf_v.at[pl.ds(j * 128, 128)],
                                 acc_sh.at[idxA[j]], sem, add=True)
                for j in range(nchunk)]
        for d in pend:
            d.wait()
        plsc.subcore_barrier()
        pltpu.sync_copy(acc_sh.at[pl.ds(s * stripe, stripe)],
                        cntA_hbm.at[c, pl.ds(s * stripe, stripe)])
        pltpu.sync_copy(zeros_hbm, acc_sh.at[pl.ds(s * stripe, stripe)])
        plsc.subcore_barrier()
        pend = [pltpu.async_copy(buf_v.at[pl.ds(j * 128, 128)],
                                 acc_sh.at[idxB[j]], sem, add=True)
                for j in range(nchunk)]
        for d in pend:
            d.wait()
        plsc.subcore_barrier()
        pltpu.sync_copy(acc_sh.at[pl.ds(s * stripe, stripe)],
                        cntB_hbm.at[c, pl.ds(s * stripe, stripe)])
        pend = [pltpu.async_copy(table_hbm.at[gidx_v.at[j]],
                                 buf_v.at[pl.ds(j * 128, 128)], sem)
                for j in range(nchunk)]
        for d in pend:
            d.wait()
        pltpu.sync_copy(buf_v, xs_hbm.at[pl.ds(wid * rpt, rpt)])

    return pre_k


@functools.lru_cache(maxsize=None)
def _make_sc_scatter2(n_rows_pad, n_out):
    rpt = n_rows_pad // NW
    nchunk = rpt // 128
    stripe = n_out // NSUB
    assert stripe % 8 == 0
    mesh = plsc.VectorSubcoreMesh(core_axis_name="c", subcore_axis_name="s",
                                  num_cores=NCORES, num_subcores=NSUB)

    @functools.partial(
        pl.kernel,
        out_type=(jax.ShapeDtypeStruct((NCORES, n_out, D), jnp.float32),
                  jax.ShapeDtypeStruct((NCORES, n_out, D), jnp.float32)),
        mesh=mesh,
        scratch_types=(
            [pltpu.VMEM((128,), jnp.int32) for _ in range(2 * nchunk)]
            + [pltpu.VMEM((rpt, D), jnp.float32),
               pltpu.VMEM_SHARED((n_out, D), jnp.float32),
               pltpu.SemaphoreType.DMA]
        ),
    )
    def scatter2_k(vals_hbm, idxA_hbm, idxB_hbm, zeros_hbm,
                   outA_hbm, outB_hbm, *scratch):
        idxA = scratch[:nchunk]
        idxB = scratch[nchunk:2 * nchunk]
        vals_v = scratch[2 * nchunk]
        acc_sh = scratch[2 * nchunk + 1]
        sem = scratch[2 * nchunk + 2]
        c = lax.axis_index("c")
        s = lax.axis_index("s")
        wid = c * NSUB + s
        pend = [pltpu.async_copy(
            zeros_hbm, acc_sh.at[pl.ds(s * stripe, stripe)], sem)]
        pend.append(pltpu.async_copy(
            vals_hbm.at[pl.ds(wid * rpt, rpt)], vals_v, sem))
        for j in range(nchunk):
            pend.append(pltpu.async_copy(
                idxA_hbm.at[wid * nchunk + j], idxA[j], sem))
            pend.append(pltpu.async_copy(
                idxB_hbm.at[wid * nchunk + j], idxB[j], sem))
        for d in pend:
            d.wait()
        plsc.subcore_barrier()
        pend = [pltpu.async_copy(vals_v.at[pl.ds(j * 128, 128)],
                                 acc_sh.at[idxA[j]], sem, add=True)
                for j in range(nchunk)]
        for d in pend:
            d.wait()
        plsc.subcore_barrier()
        pltpu.sync_copy(acc_sh.at[pl.ds(s * stripe, stripe)],
                        outA_hbm.at[c, pl.ds(s * stripe, stripe)])
        pltpu.sync_copy(zeros_hbm, acc_sh.at[pl.ds(s * stripe, stripe)])
        plsc.subcore_barrier()
        pend = [pltpu.async_copy(vals_v.at[pl.ds(j * 128, 128)],
                                 acc_sh.at[idxB[j]], sem, add=True)
                for j in range(nchunk)]
        for d in pend:
            d.wait()
        plsc.subcore_barrier()
        pltpu.sync_copy(acc_sh.at[pl.ds(s * stripe, stripe)],
                        outB_hbm.at[c, pl.ds(s * stripe, stripe)])

    return scatter2_k


@functools.lru_cache(maxsize=None)
def _make_sc_graph():
    rpt = N_RPT
    nchunk = rpt // 128
    stripe = G_SOUT // NSUB
    mesh = plsc.VectorSubcoreMesh(core_axis_name="c", subcore_axis_name="s",
                                  num_cores=NCORES, num_subcores=NSUB)

    @functools.partial(
        pl.kernel,
        out_type=(jax.ShapeDtypeStruct((NCORES, G_SOUT, D), jnp.float32),
                  jax.ShapeDtypeStruct((NCORES, G_SOUT, D), jnp.float32)),
        mesh=mesh,
        scratch_types=(
            [pltpu.VMEM((128,), jnp.int32) for _ in range(nchunk)]
            + [pltpu.VMEM((rpt, D), jnp.float32),
               pltpu.VMEM((rpt, D), jnp.float32),
               pltpu.VMEM_SHARED((G_SOUT, D), jnp.float32),
               pltpu.VMEM_SHARED((G_SOUT, D), jnp.float32),
               pltpu.SemaphoreType.DMA]
        ),
    )
    def graph_k(vals_hbm, idx_hbm, ones_hbm, zeros_hbm,
                gp_hbm, gcnt_hbm, *scratch):
        idxs = scratch[:nchunk]
        vals_v = scratch[nchunk]
        ones_v = scratch[nchunk + 1]
        accP = scratch[nchunk + 2]
        accC = scratch[nchunk + 3]
        sem = scratch[nchunk + 4]
        c = lax.axis_index("c")
        s = lax.axis_index("s")
        wid = c * NSUB + s
        pend = [pltpu.async_copy(
            zeros_hbm, accP.at[pl.ds(s * stripe, stripe)], sem),
            pltpu.async_copy(
                zeros_hbm, accC.at[pl.ds(s * stripe, stripe)], sem),
            pltpu.async_copy(
                vals_hbm.at[pl.ds(wid * rpt, rpt)], vals_v, sem),
            pltpu.async_copy(ones_hbm, ones_v, sem)]
        for j in range(nchunk):
            pend.append(pltpu.async_copy(
                idx_hbm.at[wid * nchunk + j], idxs[j], sem))
        for d in pend:
            d.wait()
        plsc.subcore_barrier()
        pend = []
        for j in range(nchunk):
            pend.append(pltpu.async_copy(
                vals_v.at[pl.ds(j * 128, 128)], accP.at[idxs[j]], sem,
                add=True))
            pend.append(pltpu.async_copy(
                ones_v.at[pl.ds(j * 128, 128)], accC.at[idxs[j]], sem,
                add=True))
        for d in pend:
            d.wait()
        plsc.subcore_barrier()
        pltpu.sync_copy(accP.at[pl.ds(s * stripe, stripe)],
                        gp_hbm.at[c, pl.ds(s * stripe, stripe)])
        pltpu.sync_copy(accC.at[pl.ds(s * stripe, stripe)],
                        gcnt_hbm.at[c, pl.ds(s * stripe, stripe)])

    return graph_k


@functools.lru_cache(maxsize=None)
def _make_sc_scatter(n_rows_pad, n_out):
    rpt = n_rows_pad // NW
    nchunk = rpt // 128
    stripe = n_out // NSUB
    assert stripe % 8 == 0
    mesh = plsc.VectorSubcoreMesh(core_axis_name="c", subcore_axis_name="s",
                                  num_cores=NCORES, num_subcores=NSUB)

    @functools.partial(
        pl.kernel,
        out_type=jax.ShapeDtypeStruct((NCORES, n_out, D), jnp.float32),
        mesh=mesh,
        scratch_types=(
            [pltpu.VMEM((128,), jnp.int32) for _ in range(nchunk)]
            + [pltpu.VMEM((rpt, D), jnp.float32),
               pltpu.VMEM_SHARED((n_out, D), jnp.float32)]
        ),
    )
    def scatter_k(vals_hbm, idx2d_hbm, zeros_hbm, out_hbm, *scratch):
        idx_vs = scratch[:nchunk]
        vals_v = scratch[nchunk]
        acc_sh = scratch[nchunk + 1]
        c = lax.axis_index("c")
        s = lax.axis_index("s")
        wid = c * NSUB + s
        pltpu.sync_copy(zeros_hbm, acc_sh.at[pl.ds(s * stripe, stripe)])
        pltpu.sync_copy(vals_hbm.at[pl.ds(wid * rpt, rpt)], vals_v)
        for j in range(nchunk):
            pltpu.sync_copy(idx2d_hbm.at[wid * nchunk + j], idx_vs[j])
        plsc.subcore_barrier()
        for j in range(nchunk):
            pltpu.sync_copy(
                vals_v.at[pl.ds(j * 128, 128)],
                acc_sh.at[idx_vs[j]], add=True)
        plsc.subcore_barrier()
        pltpu.sync_copy(
            acc_sh.at[pl.ds(s * stripe, stripe)],
            out_hbm.at[c, pl.ds(s * stripe, stripe)])

    return scatter_k


def _make_tc_edge_msg(in_ch, out_ch, kk, e_blk):
    n_steps = E_PAD // e_blk
    io = in_ch * out_ch
    n_groups = io // D
    n_fold = D // out_ch

    def body(ea_ref, xs_ref, w1_ref, b1_ref, w2_ref, b2r_ref, rx_ref, o_ref):
        h = jnp.maximum(
            jnp.dot(ea_ref[...], w1_ref[...],
                    preferred_element_type=jnp.float32) + b1_ref[...], 0.0)
        t = jnp.dot(h.astype(jnp.bfloat16), w2_ref[...],
                    preferred_element_type=jnp.float32)
        xs = xs_ref[:, 0:in_ch]
        f = jnp.dot(xs, rx_ref[...], preferred_element_type=jnp.float32)
        p = f * t
        g = p[:, 0:D]
        for c in range(1, n_groups):
            g = g + p[:, c * D:(c + 1) * D]
        acc = g[:, 0:out_ch]
        for j in range(1, n_fold):
            acc = acc + g[:, j * out_ch:(j + 1) * out_ch]
        acc = acc + jnp.dot(xs, b2r_ref[...],
                            preferred_element_type=jnp.float32)
        row = (pl.program_id(0) * e_blk
               + lax.broadcasted_iota(jnp.int32, (e_blk, 1), 0))
        acc = jnp.where(row < N_EDGES, acc, 0.0)
        if out_ch < D:
            acc = jnp.concatenate(
                [acc, jnp.zeros((e_blk, D - out_ch), jnp.float32)], axis=1)
        o_ref[...] = acc

    def call(ea, xs, w1, b1, w2, b2):
        b2r = b2.reshape(in_ch, out_ch)
        rx = jnp.kron(jnp.eye(in_ch, dtype=jnp.float32),
                      jnp.ones((1, out_ch), jnp.float32))
        return pl.pallas_call(
            body,
            grid=(n_steps,),
            in_specs=[
                pl.BlockSpec((e_blk, NE), lambda i: (i, 0)),
                pl.BlockSpec((e_blk, D), lambda i: (i, 0)),
                pl.BlockSpec((NE, kk), lambda i: (0, 0)),
                pl.BlockSpec((1, kk), lambda i: (0, 0)),
                pl.BlockSpec((kk, io), lambda i: (0, 0)),
                pl.BlockSpec((in_ch, out_ch), lambda i: (0, 0)),
                pl.BlockSpec((in_ch, io), lambda i: (0, 0)),
            ],
            out_specs=pl.BlockSpec((e_blk, D), lambda i: (i, 0)),
            out_shape=jax.ShapeDtypeStruct((E_PAD, D), jnp.float32),
        )(ea, xs, w1, b1, w2.astype(jnp.bfloat16), b2r, rx)

    return call


def _tc_node_update(spA, spB, cntA, cntB, x, in_ch, root, bias, g, b,
                    attw, attb):
    out_ch = root.shape[1]

    def body(spA_ref, spB_ref, cA_ref, cB_ref, x_ref, root_ref, bias_ref,
             g_ref, b_ref, attw_ref, attb_ref, o_ref):
        s = jnp.concatenate(
            [spA_ref[0, 0:NS0, 0:out_ch] + spA_ref[1, 0:NS0, 0:out_ch],
             spB_ref[0, 0:NS1, 0:out_ch] + spB_ref[1, 0:NS1, 0:out_ch]],
            axis=0)
        cnt = jnp.concatenate(
            [cA_ref[0, 0:NS0, 0:1] + cA_ref[1, 0:NS0, 0:1],
             cB_ref[0, 0:NS1, 0:1] + cB_ref[1, 0:NS1, 0:1]], axis=0)
        agg = s / jnp.maximum(cnt, 1.0)
        xin = x_ref[:, 0:in_ch]
        hp = agg + jnp.dot(xin, root_ref[...],
                           preferred_element_type=jnp.float32) + bias_ref[...]
        m = jnp.mean(hp, axis=0, keepdims=True)
        hc = hp - m
        v = jnp.mean(hc * hc, axis=0, keepdims=True)
        hb = g_ref[...] * hc / jnp.sqrt(v + 1e-5) + b_ref[...]
        hr = jnp.maximum(hb, 0.0)
        a = jnp.dot(hr, attw_ref[...],
                    preferred_element_type=jnp.float32) + attb_ref[...]
        sig = 1.0 / (1.0 + jnp.exp(-a))
        out = hr * sig
        if out_ch < D:
            out = jnp.concatenate(
                [out, jnp.zeros((N_NODES, D - out_ch), jnp.float32)], axis=1)
        o_ref[...] = out

    return pl.pallas_call(
        body,
        out_shape=jax.ShapeDtypeStruct((N_NODES, D), jnp.float32),
    )(spA, spB, cntA, cntB, x, root, bias.reshape(1, out_ch),
      g.reshape(1, out_ch), b.reshape(1, out_ch), attw, attb.reshape(1, 1))


def _tc_pool_mlp(gp, gcnt, fc1w, fc1b, fc2w, fc2b):
    def body(gp_ref, gc_ref, w1_ref, b1_ref, w2_ref, b2_ref, o_ref):
        s = gp_ref[0, 0:NGRAPHS, 0:NF] + gp_ref[1, 0:NGRAPHS, 0:NF]
        cnt = gc_ref[0, 0:NGRAPHS, 0:1] + gc_ref[1, 0:NGRAPHS, 0:1]
        gmean = s / jnp.maximum(cnt, 1.0)
        h1 = jnp.maximum(
            jnp.dot(gmean, w1_ref[...],
                    preferred_element_type=jnp.float32) + b1_ref[...], 0.0)
        o_ref[...] = jnp.dot(h1, w2_ref[...],
                             preferred_element_type=jnp.float32) + b2_ref[...]

    return pl.pallas_call(
        body,
        out_shape=jax.ShapeDtypeStruct((NGRAPHS, 1), jnp.float32),
    )(gp, gcnt, fc1w, fc1b.reshape(1, -1), fc2w, fc2b.reshape(1, 1))


_edge1 = _make_tc_edge_msg(NF, 128, 128, 512)
_edge2 = _make_tc_edge_msg(128, 64, 128, 256)
_edge3 = _make_tc_edge_msg(64, 32, 64, 512)


def _nnconv_layer(edge_call, xs, x_raw, in_ch, dstA, dstB,
                  cntA, cntB, z_node, ea_p, p, pfx, bn_pfx, att_pfx):
    msg = edge_call(ea_p, xs, p[pfx + '_w1'], p[pfx + '_b1'].reshape(1, -1),
                    p[pfx + '_w2'], p[pfx + '_b2'].reshape(1, -1))
    spA, spB = _make_sc_scatter2(E_PAD, NOUT_SPLIT)(msg, dstA, dstB, z_node)
    cpfx = pfx.replace('nn', 'conv')
    return _tc_node_update(spA, spB, cntA, cntB, x_raw, in_ch,
                           p[cpfx + '_root'], p[cpfx + '_bias'],
                           p[bn_pfx + '_g'], p[bn_pfx + '_b'],
                           p[att_pfx + '_w'], p[att_pfx + '_b'])


def kernel(x, edge_index, edge_attr, batch, params):
    p = params
    src = edge_index[0]
    dst = edge_index[1]

    epad = E_PAD - N_EDGES
    src3 = jnp.pad(src, (0, epad)).reshape(NW, E_RPT // 128, 128)
    dst_p = jnp.pad(dst, (0, epad))
    emask_b = jnp.arange(E_PAD, dtype=jnp.int32) < N_EDGES
    dstA = jnp.where(dst_p < NS0, dst_p, NS0).reshape(E_PAD // 128, 128)
    dstB = jnp.where(dst_p >= NS0, dst_p - NS0, NS0).reshape(E_PAD // 128, 128)
    cdstA = jnp.where(emask_b & (dst_p < NS0),
                      dst_p, NS0).reshape(E_PAD // 128, 128)
    cdstB = jnp.where(emask_b & (dst_p >= NS0),
                      dst_p - NS0, NS0).reshape(E_PAD // 128, 128)
    ea_p = jnp.pad(edge_attr, ((0, epad), (0, 0)))

    npad = N_PAD - N_NODES
    batch2d = jnp.pad(batch, (0, npad),
                      constant_values=NGRAPHS).reshape(N_PAD // 128, 128)

    ones_e = jnp.ones((E_RPT, D), jnp.float32)
    ones_n = jnp.ones((N_RPT, D), jnp.float32)
    z_node = jnp.zeros((NOUT_SPLIT // NSUB, D), jnp.float32)
    z_g = jnp.zeros((G_SOUT // NSUB, D), jnp.float32)

    x_p128 = jnp.pad(x, ((0, 0), (0, D - NF)))
    xs1, cntA, cntB = _make_sc_pre()(x_p128, src3, cdstA, cdstB,
                                     ones_e, z_node)

    h1 = _nnconv_layer(_edge1, xs1, x, NF, dstA, dstB, cntA, cntB,
                       z_node, ea_p, p, 'nn1', 'bn1', 'att1')
    xs2 = _make_sc_gather(E_PAD)(h1, src3)
    h2 = _nnconv_layer(_edge2, xs2, h1, 128, dstA, dstB, cntA, cntB,
                       z_node, ea_p, p, 'nn2', 'bn2', 'att2')
    xs3 = _make_sc_gather(E_PAD)(h2, src3)
    h3 = _nnconv_layer(_edge3, xs3, h2, 64, dstA, dstB, cntA, cntB,
                       z_node, ea_p, p, 'nn3', 'bn3', 'att3')

    h3_p = jnp.pad(h3, ((0, npad), (0, 0)))
    gp, gcnt = _make_sc_graph()(h3_p, batch2d, ones_n, z_g)
    return _tc_pool_mlp(gp, gcnt, p['fc1_w'], p['fc1_b'],
                        p['fc2_w'], p['fc2_b'])
